# Initial kernel scaffold; baseline (speedup 1.0000x reference)
import jax
import jax.numpy as jnp
from jax.experimental import pallas as pl


def kernel(user_input, item_input, daytime_input, weekend_input, year_input, user_table, item_table, daytime_table, weekend_table, year_table, user_time_bias, W1, b1, W2, b2):
    raise NotImplementedError("write your pallas kernel here")


# speedup vs baseline: 1.9954x; 1.9954x over previous
import functools

import jax
import jax.numpy as jnp
from jax import lax
from jax.experimental import pallas as pl
from jax.experimental.pallas import tpu as pltpu
from jax.experimental.pallas import tpu_sc as plsc

_B = 16384
_K = 128
_NC = 2
_NS = 16
_NW = _NC * _NS
_BPW = _B // _NW
_CH = 128
_NCH = _BPW // _CH


def _sc_gather_body(uidx_hbm, iidx_hbm, didx_hbm, utab_hbm, itab_hbm,
                    bias_hbm, urows_hbm, irows_hbm, bvals_hbm,
                    uidx_v, iidx_v, didx_v, tidx_v, bbuf_v, ubuf_v, ibuf_v,
                    semgu, semgi, semb,
                    semsu0, semsu1, semsi0, semsi1, sembst):
    wid = lax.axis_index("s") * _NC + lax.axis_index("c")
    base = wid * _BPW

    pltpu.sync_copy(uidx_hbm.at[pl.ds(base, _BPW)], uidx_v)
    pltpu.sync_copy(iidx_hbm.at[pl.ds(base, _BPW)], iidx_v)
    pltpu.sync_copy(didx_hbm.at[pl.ds(base, _BPW)], didx_v)

    def _mk(j, carry):
        s = pl.ds(j * 16, 16)
        tidx_v[s] = uidx_v[s] * 3 + didx_v[s]
        return carry
    lax.fori_loop(0, _BPW // 16, _mk, 0)

    bcopies = []
    for c in range(_NCH):
        sl = pl.ds(c * _CH, _CH)
        bcopies.append(
            pltpu.async_copy(bias_hbm.at[tidx_v.at[sl]], bbuf_v.at[sl], semb))
    for bc in bcopies:
        bc.wait()
    bst = pltpu.async_copy(bbuf_v, bvals_hbm.at[pl.ds(base, _BPW)], sembst)

    semsu = (semsu0, semsu1)
    semsi = (semsi0, semsi1)
    stu = [None] * _NCH
    sti = [None] * _NCH
    for c in range(_NCH):
        b = c % 2
        if c >= 2:
            stu[c - 2].wait()
            sti[c - 2].wait()
        sl = pl.ds(c * _CH, _CH)
        gu = pltpu.async_copy(utab_hbm.at[uidx_v.at[sl]], ubuf_v.at[b], semgu)
        gi = pltpu.async_copy(itab_hbm.at[iidx_v.at[sl]], ibuf_v.at[b], semgi)
        gu.wait()
        gi.wait()
        osl = pl.ds(base + c * _CH, _CH)
        stu[c] = pltpu.async_copy(ubuf_v.at[b], urows_hbm.at[osl], semsu[b])
        sti[c] = pltpu.async_copy(ibuf_v.at[b], irows_hbm.at[osl], semsi[b])
    for c in (_NCH - 2, _NCH - 1):
        stu[c].wait()
        sti[c].wait()
    bst.wait()


@functools.cache
def _make_sc_gather():
    return functools.partial(
        pl.kernel,
        out_type=[
                jax.ShapeDtypeStruct((_B, _K), jnp.float32),
            jax.ShapeDtypeStruct((_B, _K), jnp.float32),
            jax.ShapeDtypeStruct((_B,), jnp.float32),
        ],
        mesh=plsc.VectorSubcoreMesh(core_axis_name="c", subcore_axis_name="s"),
        scratch_types=[
            pltpu.VMEM((_BPW,), jnp.int32),
            pltpu.VMEM((_BPW,), jnp.int32),
            pltpu.VMEM((_BPW,), jnp.int32),
            pltpu.VMEM((_BPW,), jnp.int32),
            pltpu.VMEM((_BPW,), jnp.float32),
            pltpu.VMEM((2, _CH, _K), jnp.float32),
            pltpu.VMEM((2, _CH, _K), jnp.float32),
            pltpu.SemaphoreType.DMA,
            pltpu.SemaphoreType.DMA,
            pltpu.SemaphoreType.DMA,
            pltpu.SemaphoreType.DMA,
            pltpu.SemaphoreType.DMA,
            pltpu.SemaphoreType.DMA,
            pltpu.SemaphoreType.DMA,
            pltpu.SemaphoreType.DMA,
        ],
    )(_sc_gather_body)


_BLK = 2048


def _tc_mlp_body(u_ref, i_ref, d_ref, w_ref, y_ref, bias_ref,
                 dt_ref, wk_ref, yr_ref, w1a_ref, w1t_ref, b1_ref,
                 w2_ref, b2_ref, out_ref):
    inter = u_ref[...] * i_ref[...]
    w1t = w1t_ref[...]
    pd = jnp.dot(dt_ref[...], w1t[0:10], preferred_element_type=jnp.float32)
    pw = jnp.dot(wk_ref[...], w1t[10:20], preferred_element_type=jnp.float32)
    py = jnp.dot(yr_ref[...], w1t[20:30], preferred_element_type=jnp.float32)
    d = d_ref[...]
    w = w_ref[...]
    y = y_ref[...]
    dh = (lax.broadcasted_iota(jnp.int32, (_BLK, 3), 1) == d).astype(jnp.float32)
    wh = (lax.broadcasted_iota(jnp.int32, (_BLK, 2), 1) == w).astype(jnp.float32)
    yh = (lax.broadcasted_iota(jnp.int32, (_BLK, 20), 1) == y).astype(jnp.float32)
    acc = jnp.dot(inter, w1a_ref[...], preferred_element_type=jnp.float32)
    acc = acc + jnp.dot(dh, pd, preferred_element_type=jnp.float32)
    acc = acc + jnp.dot(wh, pw, preferred_element_type=jnp.float32)
    acc = acc + jnp.dot(yh, py, preferred_element_type=jnp.float32)
    h = jnp.maximum(acc + b1_ref[...], 0.0)
    out = jnp.dot(h, w2_ref[...], preferred_element_type=jnp.float32)
    out_ref[...] = out + b2_ref[...] + bias_ref[...]


def _tc_mlp(urows, irows, d2, w2d, y2, bias2, dt, wk, yr, w1a, w1t, b1, w2, b2):
    grid = (_B // _BLK,)
    row_spec = pl.BlockSpec((_BLK, _K), lambda i: (i, 0))
    col_spec = pl.BlockSpec((_BLK, 1), lambda i: (i, 0))

    def full(a):
        return pl.BlockSpec(a.shape, lambda i: tuple(0 for _ in a.shape))

    return pl.pallas_call(
        _tc_mlp_body,
        grid=grid,
        in_specs=[row_spec, row_spec, col_spec, col_spec, col_spec, col_spec,
                  full(dt), full(wk), full(yr), full(w1a), full(w1t),
                  full(b1), full(w2), full(b2)],
        out_specs=col_spec,
        out_shape=jax.ShapeDtypeStruct((_B, 1), jnp.float32),
    )(urows, irows, d2, w2d, y2, bias2, dt, wk, yr, w1a, w1t, b1, w2, b2)


def kernel(user_input, item_input, daytime_input, weekend_input, year_input,
           user_table, item_table, daytime_table, weekend_table, year_table,
           user_time_bias, W1, b1, W2, b2):
    bias_flat = user_time_bias.reshape(-1)
    urows, irows, bvals = _make_sc_gather()(
        user_input.astype(jnp.int32), item_input.astype(jnp.int32),
        daytime_input.astype(jnp.int32), user_table, item_table, bias_flat)
    out = _tc_mlp(
        urows, irows,
        daytime_input.astype(jnp.int32).reshape(_B, 1),
        weekend_input.astype(jnp.int32).reshape(_B, 1),
        year_input.astype(jnp.int32).reshape(_B, 1),
        bvals.reshape(_B, 1),
        daytime_table, weekend_table, year_table,
        W1[0:_K], W1[_K:], b1.reshape(1, -1), W2, b2.reshape(1, 1))
    return out.reshape(_B)


# speedup vs baseline: 3.9639x; 1.9865x over previous
import functools

import jax
import jax.numpy as jnp
from jax import lax
from jax.experimental import pallas as pl
from jax.experimental.pallas import tpu as pltpu
from jax.experimental.pallas import tpu_sc as plsc

_B = 16384
_K = 128
_NC = 2
_NS = 16
_NW = _NC * _NS
_BPW = _B // _NW
_CH = 128
_NCH = _BPW // _CH


def _sc_gather_body(uidx_hbm, iidx_hbm, didx_hbm, utab_hbm, itab_hbm,
                    c0_hbm, c1_hbm, c2_hbm,
                    urows_hbm, irows_hbm, bvals_hbm,
                    uidx_v, iidx_v, didx_v, b0_v, b1_v, b2_v, bbuf_v,
                    ubuf_v, ibuf_v,
                    semgu, semgi, semb,
                    semsu0, semsu1, semsi0, semsi1, sembst):
    wid = lax.axis_index("s") * _NC + lax.axis_index("c")
    base = wid * _BPW

    pltpu.sync_copy(uidx_hbm.at[pl.ds(base, _BPW)], uidx_v)
    pltpu.sync_copy(iidx_hbm.at[pl.ds(base, _BPW)], iidx_v)
    pltpu.sync_copy(didx_hbm.at[pl.ds(base, _BPW)], didx_v)

    bcopies = []
    for c in range(_NCH):
        sl = pl.ds(c * _CH, _CH)
        isl = uidx_v.at[sl]
        bcopies.append(pltpu.async_copy(c0_hbm.at[isl], b0_v.at[sl], semb))
        bcopies.append(pltpu.async_copy(c1_hbm.at[isl], b1_v.at[sl], semb))
        bcopies.append(pltpu.async_copy(c2_hbm.at[isl], b2_v.at[sl], semb))

    semsu = (semsu0, semsu1)
    semsi = (semsi0, semsi1)
    stu = [None] * _NCH
    sti = [None] * _NCH
    for c in range(_NCH):
        b = c % 2
        if c >= 2:
            stu[c - 2].wait()
            sti[c - 2].wait()
        sl = pl.ds(c * _CH, _CH)
        gu = pltpu.async_copy(utab_hbm.at[uidx_v.at[sl]], ubuf_v.at[b], semgu)
        gi = pltpu.async_copy(itab_hbm.at[iidx_v.at[sl]], ibuf_v.at[b], semgi)
        gu.wait()
        gi.wait()
        osl = pl.ds(base + c * _CH, _CH)
        stu[c] = pltpu.async_copy(ubuf_v.at[b], urows_hbm.at[osl], semsu[b])
        sti[c] = pltpu.async_copy(ibuf_v.at[b], irows_hbm.at[osl], semsi[b])

    for bc in bcopies:
        bc.wait()

    def _sel(j, carry):
        s = pl.ds(j * 16, 16)
        d = didx_v[s]
        bbuf_v[s] = jnp.where(d == 0, b0_v[s],
                              jnp.where(d == 1, b1_v[s], b2_v[s]))
        return carry
    lax.fori_loop(0, _BPW // 16, _sel, 0)
    bst = pltpu.async_copy(bbuf_v, bvals_hbm.at[pl.ds(base, _BPW)], sembst)

    for c in (_NCH - 2, _NCH - 1):
        stu[c].wait()
        sti[c].wait()
    bst.wait()


@functools.cache
def _make_sc_gather():
    return functools.partial(
        pl.kernel,
        out_type=[
            jax.ShapeDtypeStruct((_B, _K), jnp.float32),
            jax.ShapeDtypeStruct((_B, _K), jnp.float32),
            jax.ShapeDtypeStruct((_B,), jnp.float32),
        ],
        mesh=plsc.VectorSubcoreMesh(core_axis_name="c", subcore_axis_name="s"),
        scratch_types=[
            pltpu.VMEM((_BPW,), jnp.int32),
            pltpu.VMEM((_BPW,), jnp.int32),
            pltpu.VMEM((_BPW,), jnp.int32),
            pltpu.VMEM((_BPW,), jnp.float32),
            pltpu.VMEM((_BPW,), jnp.float32),
            pltpu.VMEM((_BPW,), jnp.float32),
            pltpu.VMEM((_BPW,), jnp.float32),
            pltpu.VMEM((2, _CH, _K), jnp.float32),
            pltpu.VMEM((2, _CH, _K), jnp.float32),
            pltpu.SemaphoreType.DMA,
            pltpu.SemaphoreType.DMA,
            pltpu.SemaphoreType.DMA,
            pltpu.SemaphoreType.DMA,
            pltpu.SemaphoreType.DMA,
            pltpu.SemaphoreType.DMA,
            pltpu.SemaphoreType.DMA,
            pltpu.SemaphoreType.DMA,
        ],
    )(_sc_gather_body)


_BLK = 2048


def _tc_mlp_body(u_ref, i_ref, code_ref,
                 dt_ref, wk_ref, yr_ref, w1a_ref, w1t_ref, b1_ref,
                 w2_ref, b2_ref, out_ref):
    inter = u_ref[...] * i_ref[...]
    w1t = w1t_ref[...]
    pd = jnp.dot(dt_ref[...], w1t[0:10], preferred_element_type=jnp.float32)
    pw = jnp.dot(wk_ref[...], w1t[10:20], preferred_element_type=jnp.float32)
    py = jnp.dot(yr_ref[...], w1t[20:30], preferred_element_type=jnp.float32)
    code = code_ref[...]
    d = code % 3
    w = (code // 3) % 2
    y = code // 6
    dh = (lax.broadcasted_iota(jnp.int32, (_BLK, 3), 1) == d).astype(jnp.float32)
    wh = (lax.broadcasted_iota(jnp.int32, (_BLK, 2), 1) == w).astype(jnp.float32)
    yh = (lax.broadcasted_iota(jnp.int32, (_BLK, 20), 1) == y).astype(jnp.float32)
    acc = jnp.dot(inter, w1a_ref[...], preferred_element_type=jnp.float32)
    acc = acc + jnp.dot(dh, pd, preferred_element_type=jnp.float32)
    acc = acc + jnp.dot(wh, pw, preferred_element_type=jnp.float32)
    acc = acc + jnp.dot(yh, py, preferred_element_type=jnp.float32)
    h = jnp.maximum(acc + b1_ref[...], 0.0)
    out = jnp.dot(h, w2_ref[...], preferred_element_type=jnp.float32)
    out_ref[...] = out + b2_ref[...]


def _tc_mlp(urows, irows, code2, dt, wk, yr, w1a, w1t, b1, w2, b2):
    grid = (_B // _BLK,)
    row_spec = pl.BlockSpec((_BLK, _K), lambda i: (i, 0))
    col_spec = pl.BlockSpec((_BLK, 1), lambda i: (i, 0))

    def full(a):
        return pl.BlockSpec(a.shape, lambda i: tuple(0 for _ in a.shape))

    return pl.pallas_call(
        _tc_mlp_body,
        grid=grid,
        in_specs=[row_spec, row_spec, col_spec,
                  full(dt), full(wk), full(yr), full(w1a), full(w1t),
                  full(b1), full(w2), full(b2)],
        out_specs=col_spec,
        out_shape=jax.ShapeDtypeStruct((_B, 1), jnp.float32),
    )(urows, irows, code2, dt, wk, yr, w1a, w1t, b1, w2, b2)


def kernel(user_input, item_input, daytime_input, weekend_input, year_input,
           user_table, item_table, daytime_table, weekend_table, year_table,
           user_time_bias, W1, b1, W2, b2):
    ui = user_input.astype(jnp.int32)
    di = daytime_input.astype(jnp.int32)
    urows, irows, bvals = _make_sc_gather()(
        ui, item_input.astype(jnp.int32), di, user_table, item_table,
        user_time_bias[:, 0], user_time_bias[:, 1], user_time_bias[:, 2])
    code = di + 3 * weekend_input.astype(jnp.int32) \
        + 6 * year_input.astype(jnp.int32)
    out = _tc_mlp(
        urows, irows, code.reshape(_B, 1),
        daytime_table, weekend_table, year_table,
        W1[0:_K], W1[_K:], b1.reshape(1, -1), W2, b2.reshape(1, 1))
    return out.reshape(_B) + bvals


# speedup vs baseline: 4.0586x; 1.0239x over previous
import functools

import jax
import jax.numpy as jnp
from jax import lax
from jax.experimental import pallas as pl
from jax.experimental.pallas import tpu as pltpu
from jax.experimental.pallas import tpu_sc as plsc

_B = 16384
_K = 128
_NC = 2
_NS = 16
_NW = _NC * _NS
_BPW = _B // _NW
_CH = 128
_NCH = _BPW // _CH


def _sc_gather_body(uidx_hbm, iidx_hbm, didx_hbm, utab_hbm, itab_hbm,
                    c0_hbm, c1_hbm, c2_hbm,
                    urows_hbm, irows_hbm, bvals_hbm,
                    uidx_v, iidx_v, didx_v, b0_v, b1_v, b2_v, bbuf_v,
                    ubuf_v, ibuf_v,
                    semgu, semgi, semb,
                    semsu0, semsu1, semsi0, semsi1, sembst):
    wid = lax.axis_index("s") * _NC + lax.axis_index("c")
    base = wid * _BPW

    pltpu.sync_copy(uidx_hbm.at[pl.ds(base, _BPW)], uidx_v)
    pltpu.sync_copy(iidx_hbm.at[pl.ds(base, _BPW)], iidx_v)
    pltpu.sync_copy(didx_hbm.at[pl.ds(base, _BPW)], didx_v)

    bcopies = []
    for c in range(_NCH):
        sl = pl.ds(c * _CH, _CH)
        isl = uidx_v.at[sl]
        bcopies.append(pltpu.async_copy(c0_hbm.at[isl], b0_v.at[sl], semb))
        bcopies.append(pltpu.async_copy(c1_hbm.at[isl], b1_v.at[sl], semb))
        bcopies.append(pltpu.async_copy(c2_hbm.at[isl], b2_v.at[sl], semb))

    semsu = (semsu0, semsu1)
    semsi = (semsi0, semsi1)
    stu = [None] * _NCH
    sti = [None] * _NCH
    for c in range(_NCH):
        b = c % 2
        if c >= 2:
            stu[c - 2].wait()
            sti[c - 2].wait()
        sl = pl.ds(c * _CH, _CH)
        gu = pltpu.async_copy(utab_hbm.at[uidx_v.at[sl]], ubuf_v.at[b], semgu)
        gi = pltpu.async_copy(itab_hbm.at[iidx_v.at[sl]], ibuf_v.at[b], semgi)
        gu.wait()
        gi.wait()
        osl = pl.ds(base + c * _CH, _CH)
        stu[c] = pltpu.async_copy(ubuf_v.at[b], urows_hbm.at[osl], semsu[b])
        sti[c] = pltpu.async_copy(ibuf_v.at[b], irows_hbm.at[osl], semsi[b])

    for bc in bcopies:
        bc.wait()

    def _sel(j, carry):
        s = pl.ds(j * 16, 16)
        d = didx_v[s]
        bbuf_v[s] = jnp.where(d == 0, b0_v[s],
                              jnp.where(d == 1, b1_v[s], b2_v[s]))
        return carry
    lax.fori_loop(0, _BPW // 16, _sel, 0)
    bst = pltpu.async_copy(bbuf_v, bvals_hbm.at[pl.ds(base, _BPW)], sembst)

    for c in (_NCH - 2, _NCH - 1):
        stu[c].wait()
        sti[c].wait()
    bst.wait()


@functools.cache
def _make_sc_gather():
    return functools.partial(
        pl.kernel,
        out_type=[
            jax.ShapeDtypeStruct((_B, _K), jnp.float32),
            jax.ShapeDtypeStruct((_B, _K), jnp.float32),
            jax.ShapeDtypeStruct((_B,), jnp.float32),
        ],
        mesh=plsc.VectorSubcoreMesh(core_axis_name="c", subcore_axis_name="s"),
        scratch_types=[
            pltpu.VMEM((_BPW,), jnp.int32),
            pltpu.VMEM((_BPW,), jnp.int32),
            pltpu.VMEM((_BPW,), jnp.int32),
            pltpu.VMEM((_BPW,), jnp.float32),
            pltpu.VMEM((_BPW,), jnp.float32),
            pltpu.VMEM((_BPW,), jnp.float32),
            pltpu.VMEM((_BPW,), jnp.float32),
            pltpu.VMEM((2, _CH, _K), jnp.float32),
            pltpu.VMEM((2, _CH, _K), jnp.float32),
            pltpu.SemaphoreType.DMA,
            pltpu.SemaphoreType.DMA,
            pltpu.SemaphoreType.DMA,
            pltpu.SemaphoreType.DMA,
            pltpu.SemaphoreType.DMA,
            pltpu.SemaphoreType.DMA,
            pltpu.SemaphoreType.DMA,
            pltpu.SemaphoreType.DMA,
        ],
    )(_sc_gather_body)


_BLK = 2048


def _tc_mlp_body(u_ref, i_ref, code_ref, bias_ref,
                 dt_ref, wk_ref, yr_ref, w1a_ref, w1t_ref, b1_ref,
                 w2_ref, b2_ref, out_ref):
    inter = u_ref[...] * i_ref[...]
    w1t = w1t_ref[...]
    pd = jnp.dot(dt_ref[...], w1t[0:10], preferred_element_type=jnp.float32)
    pw = jnp.dot(wk_ref[...], w1t[10:20], preferred_element_type=jnp.float32)
    py = jnp.dot(yr_ref[...], w1t[20:30], preferred_element_type=jnp.float32)
    code = code_ref[...].reshape(_BLK, 1)
    d = code % 3
    w = (code // 3) % 2
    y = code // 6
    dh = (lax.broadcasted_iota(jnp.int32, (_BLK, 3), 1) == d).astype(jnp.float32)
    wh = (lax.broadcasted_iota(jnp.int32, (_BLK, 2), 1) == w).astype(jnp.float32)
    yh = (lax.broadcasted_iota(jnp.int32, (_BLK, 20), 1) == y).astype(jnp.float32)
    acc = jnp.dot(inter, w1a_ref[...], preferred_element_type=jnp.float32)
    acc = acc + jnp.dot(dh, pd, preferred_element_type=jnp.float32)
    acc = acc + jnp.dot(wh, pw, preferred_element_type=jnp.float32)
    acc = acc + jnp.dot(yh, py, preferred_element_type=jnp.float32)
    h = jnp.maximum(acc + b1_ref[...], 0.0)
    out = jnp.dot(h, w2_ref[...], preferred_element_type=jnp.float32)
    out_ref[...] = (out + b2_ref[...]).reshape(_BLK) + bias_ref[...]


def _tc_mlp(urows, irows, code, bvals, dt, wk, yr, w1a, w1t, b1, w2, b2):
    grid = (_B // _BLK,)
    row_spec = pl.BlockSpec((_BLK, _K), lambda i: (i, 0))
    vec_spec = pl.BlockSpec((_BLK,), lambda i: (i,))

    def full(a):
        return pl.BlockSpec(a.shape, lambda i: tuple(0 for _ in a.shape))

    return pl.pallas_call(
        _tc_mlp_body,
        grid=grid,
        in_specs=[row_spec, row_spec, vec_spec, vec_spec,
                  full(dt), full(wk), full(yr), full(w1a), full(w1t),
                  full(b1), full(w2), full(b2)],
        out_specs=vec_spec,
        out_shape=jax.ShapeDtypeStruct((_B,), jnp.float32),
    )(urows, irows, code, bvals, dt, wk, yr, w1a, w1t, b1, w2, b2)


def kernel(user_input, item_input, daytime_input, weekend_input, year_input,
           user_table, item_table, daytime_table, weekend_table, year_table,
           user_time_bias, W1, b1, W2, b2):
    ui = user_input.astype(jnp.int32)
    di = daytime_input.astype(jnp.int32)
    urows, irows, bvals = _make_sc_gather()(
        ui, item_input.astype(jnp.int32), di, user_table, item_table,
        user_time_bias[:, 0], user_time_bias[:, 1], user_time_bias[:, 2])
    code = di + 3 * weekend_input.astype(jnp.int32) \
        + 6 * year_input.astype(jnp.int32)
    return _tc_mlp(
        urows, irows, code, bvals,
        daytime_table, weekend_table, year_table,
        W1[0:_K], W1[_K:], b1.reshape(1, -1), W2, b2.reshape(1, 1))


# speedup vs baseline: 5.0590x; 1.2465x over previous
import functools

import jax
import jax.numpy as jnp
from jax import lax
from jax.experimental import pallas as pl
from jax.experimental.pallas import tpu as pltpu
from jax.experimental.pallas import tpu_sc as plsc

_B = 16384
_K = 128
_NC = 2
_NS = 16
_NW = _NC * _NS
_BPW = _B // _NW
_CH = 128
_NCH = _BPW // _CH


def _sc_gather_body(uidx_hbm, iidx_hbm, didx_hbm, utab_hbm, itab_hbm,
                    c0_hbm, c1_hbm, c2_hbm,
                    urows_hbm, irows_hbm, bvals_hbm,
                    uidx_v, iidx_v, didx_v, b0_v, b1_v, b2_v, bbuf_v,
                    ubuf_v, ibuf_v,
                    semgu, semgi, semb,
                    semsu0, semsu1, semsi0, semsi1, sembst):
    wid = lax.axis_index("s") * _NC + lax.axis_index("c")
    base = wid * _BPW

    pltpu.sync_copy(uidx_hbm.at[pl.ds(base, _BPW)], uidx_v)
    pltpu.sync_copy(iidx_hbm.at[pl.ds(base, _BPW)], iidx_v)
    pltpu.sync_copy(didx_hbm.at[pl.ds(base, _BPW)], didx_v)

    bcopies = []
    for c in range(_NCH):
        sl = pl.ds(c * _CH, _CH)
        isl = uidx_v.at[sl]
        bcopies.append(pltpu.async_copy(c0_hbm.at[isl], b0_v.at[sl], semb))
        bcopies.append(pltpu.async_copy(c1_hbm.at[isl], b1_v.at[sl], semb))
        bcopies.append(pltpu.async_copy(c2_hbm.at[isl], b2_v.at[sl], semb))

    semsu = (semsu0, semsu1)
    semsi = (semsi0, semsi1)
    stu = [None] * _NCH
    sti = [None] * _NCH
    for c in range(_NCH):
        b = c % 2
        if c >= 2:
            stu[c - 2].wait()
            sti[c - 2].wait()
        sl = pl.ds(c * _CH, _CH)
        gu = pltpu.async_copy(utab_hbm.at[uidx_v.at[sl]], ubuf_v.at[b], semgu)
        gi = pltpu.async_copy(itab_hbm.at[iidx_v.at[sl]], ibuf_v.at[b], semgi)
        gu.wait()
        gi.wait()
        osl = pl.ds(base + c * _CH, _CH)
        stu[c] = pltpu.async_copy(ubuf_v.at[b], urows_hbm.at[osl], semsu[b])
        sti[c] = pltpu.async_copy(ibuf_v.at[b], irows_hbm.at[osl], semsi[b])

    for bc in bcopies:
        bc.wait()

    def _sel(j, carry):
        s = pl.ds(j * 16, 16)
        d = didx_v[s]
        bbuf_v[s] = jnp.where(d == 0, b0_v[s],
                              jnp.where(d == 1, b1_v[s], b2_v[s]))
        return carry
    lax.fori_loop(0, _BPW // 16, _sel, 0)
    bst = pltpu.async_copy(bbuf_v, bvals_hbm.at[pl.ds(base, _BPW)], sembst)

    for c in (_NCH - 2, _NCH - 1):
        stu[c].wait()
        sti[c].wait()
    bst.wait()


@functools.cache
def _make_sc_gather():
    return functools.partial(
        pl.kernel,
        out_type=[
            jax.ShapeDtypeStruct((_B, _K), jnp.float32),
            jax.ShapeDtypeStruct((_B, _K), jnp.float32),
            jax.ShapeDtypeStruct((_B,), jnp.float32),
        ],
        mesh=plsc.VectorSubcoreMesh(core_axis_name="c", subcore_axis_name="s"),
        scratch_types=[
            pltpu.VMEM((_BPW,), jnp.int32),
            pltpu.VMEM((_BPW,), jnp.int32),
            pltpu.VMEM((_BPW,), jnp.int32),
            pltpu.VMEM((_BPW,), jnp.float32),
            pltpu.VMEM((_BPW,), jnp.float32),
            pltpu.VMEM((_BPW,), jnp.float32),
            pltpu.VMEM((_BPW,), jnp.float32),
            pltpu.VMEM((2, _CH, _K), jnp.float32),
            pltpu.VMEM((2, _CH, _K), jnp.float32),
            pltpu.SemaphoreType.DMA,
            pltpu.SemaphoreType.DMA,
            pltpu.SemaphoreType.DMA,
            pltpu.SemaphoreType.DMA,
            pltpu.SemaphoreType.DMA,
            pltpu.SemaphoreType.DMA,
            pltpu.SemaphoreType.DMA,
            pltpu.SemaphoreType.DMA,
        ],
    )(_sc_gather_body)


_BLK = 2048


_NCODE = 120


def _tc_mlp_body(u_ref, i_ref, code_ref, bias_ref,
                 dt_ref, wk_ref, yr_ref, w1a_ref, w1t_ref, b1_ref,
                 w2_ref, b2_ref, out_ref):
    f32 = jnp.float32
    inter = u_ref[...] * i_ref[...]
    w1t = w1t_ref[...]
    pd = jnp.dot(dt_ref[...], w1t[0:10], preferred_element_type=f32)
    pw = jnp.dot(wk_ref[...], w1t[10:20], preferred_element_type=f32)
    py = jnp.dot(yr_ref[...], w1t[20:30], preferred_element_type=f32)
    c0 = lax.broadcasted_iota(jnp.int32, (_NCODE, 1), 0)
    e3 = (c0 % 3 == lax.broadcasted_iota(jnp.int32, (_NCODE, 3), 1)).astype(f32)
    e2 = ((c0 // 3) % 2
          == lax.broadcasted_iota(jnp.int32, (_NCODE, 2), 1)).astype(f32)
    e20 = (c0 // 6
           == lax.broadcasted_iota(jnp.int32, (_NCODE, 20), 1)).astype(f32)
    p = jnp.dot(e3, pd, preferred_element_type=f32) \
        + jnp.dot(e2, pw, preferred_element_type=f32) \
        + jnp.dot(e20, py, preferred_element_type=f32)
    code = code_ref[...]
    hot_t = (jnp.broadcast_to(code, (_NCODE, _BLK))
             == lax.broadcasted_iota(jnp.int32, (_NCODE, _BLK), 0)).astype(f32)
    acc = jnp.dot(inter, w1a_ref[...], preferred_element_type=f32)
    acc = acc + lax.dot_general(hot_t, p, (((0,), (0,)), ((), ())),
                                preferred_element_type=f32)
    h = jnp.maximum(acc + b1_ref[...], 0.0)
    w2 = w2_ref[...]
    parts = [
        lax.dot_general(w2, h[k * 128:(k + 1) * 128, :],
                        (((0,), (1,)), ((), ())), preferred_element_type=f32)
        for k in range(_BLK // 128)
    ]
    out_t = jnp.concatenate(parts, axis=0)
    out_ref[...] = out_t.reshape(_BLK) + b2_ref[0, 0] + bias_ref[...]


def _tc_mlp(urows, irows, code, bvals, dt, wk, yr, w1a, w1t, b1, w2, b2):
    grid = (_B // _BLK,)
    row_spec = pl.BlockSpec((_BLK, _K), lambda i: (i, 0))
    vec_spec = pl.BlockSpec((_BLK,), lambda i: (i,))

    def full(a):
        return pl.BlockSpec(a.shape, lambda i: tuple(0 for _ in a.shape))

    return pl.pallas_call(
        _tc_mlp_body,
        grid=grid,
        in_specs=[row_spec, row_spec, vec_spec, vec_spec,
                  full(dt), full(wk), full(yr), full(w1a), full(w1t),
                  full(b1), full(w2), full(b2)],
        out_specs=vec_spec,
        out_shape=jax.ShapeDtypeStruct((_B,), jnp.float32),
    )(urows, irows, code, bvals, dt, wk, yr, w1a, w1t, b1, w2, b2)


def kernel(user_input, item_input, daytime_input, weekend_input, year_input,
           user_table, item_table, daytime_table, weekend_table, year_table,
           user_time_bias, W1, b1, W2, b2):
    ui = user_input.astype(jnp.int32)
    di = daytime_input.astype(jnp.int32)
    urows, irows, bvals = _make_sc_gather()(
        ui, item_input.astype(jnp.int32), di, user_table, item_table,
        user_time_bias[:, 0], user_time_bias[:, 1], user_time_bias[:, 2])
    code = di + 3 * weekend_input.astype(jnp.int32) \
        + 6 * year_input.astype(jnp.int32)
    return _tc_mlp(
        urows, irows, code, bvals,
        daytime_table, weekend_table, year_table,
        W1[0:_K], W1[_K:], b1.reshape(1, -1), W2, b2.reshape(1, 1))


# speedup vs baseline: 10.9119x; 2.1569x over previous
import functools

import jax
import jax.numpy as jnp
from jax import lax
from jax.experimental import pallas as pl
from jax.experimental.pallas import tpu as pltpu
from jax.experimental.pallas import tpu_sc as plsc

_B = 16384
_K = 128
_NC = 2
_NS = 16
_NW = _NC * _NS
_BPW = _B // _NW
_CH = 128
_NCH = _BPW // _CH


def _sc_gather_body(uidx_hbm, iidx_hbm, didx_hbm, utab_hbm, itab_hbm,
                    c0_hbm, c1_hbm, c2_hbm,
                    inter_hbm, bvals_hbm,
                    uidx_v, iidx_v, didx_v, b0_v, b1_v, b2_v, bbuf_v,
                    ubuf_v, ibuf_v,
                    semgu, semgi, semb, sems0, sems1, sembst):
    wid = lax.axis_index("s") * _NC + lax.axis_index("c")
    base = wid * _BPW

    pltpu.sync_copy(uidx_hbm.at[pl.ds(base, _BPW)], uidx_v)
    pltpu.sync_copy(iidx_hbm.at[pl.ds(base, _BPW)], iidx_v)
    pltpu.sync_copy(didx_hbm.at[pl.ds(base, _BPW)], didx_v)

    bcopies = []
    for c in range(_NCH):
        sl = pl.ds(c * _CH, _CH)
        isl = uidx_v.at[sl]
        bcopies.append(pltpu.async_copy(c0_hbm.at[isl], b0_v.at[sl], semb))
        bcopies.append(pltpu.async_copy(c1_hbm.at[isl], b1_v.at[sl], semb))
        bcopies.append(pltpu.async_copy(c2_hbm.at[isl], b2_v.at[sl], semb))

    sems = (sems0, sems1)
    st = [None] * _NCH

    def fire(c):
        b = c % 2
        sl = pl.ds(c * _CH, _CH)
        return (
            pltpu.async_copy(utab_hbm.at[uidx_v.at[sl]], ubuf_v.at[b], semgu),
            pltpu.async_copy(itab_hbm.at[iidx_v.at[sl]], ibuf_v.at[b], semgi),
        )

    g = {0: fire(0)}
    for c in range(_NCH):
        b = c % 2
        g[c][0].wait()
        g[c][1].wait()
        if c + 1 < _NCH:
            if c >= 1:
                st[c - 1].wait()
            g[c + 1] = fire(c + 1)

        def _mul(r, carry):
            for j in range(_K // 16):
                s = pl.ds(j * 16, 16)
                ubuf_v[b, r, s] = ubuf_v[b, r, s] * ibuf_v[b, r, s]
            return carry
        lax.fori_loop(0, _CH, _mul, 0)
        osl = pl.ds(base + c * _CH, _CH)
        st[c] = pltpu.async_copy(ubuf_v.at[b], inter_hbm.at[osl], sems[b])

    for bc in bcopies:
        bc.wait()

    def _sel(j, carry):
        s = pl.ds(j * 16, 16)
        d = didx_v[s]
        bbuf_v[s] = jnp.where(d == 0, b0_v[s],
                              jnp.where(d == 1, b1_v[s], b2_v[s]))
        return carry
    lax.fori_loop(0, _BPW // 16, _sel, 0)
    bst = pltpu.async_copy(bbuf_v, bvals_hbm.at[pl.ds(base, _BPW)], sembst)

    st[_NCH - 2].wait()
    st[_NCH - 1].wait()
    bst.wait()


@functools.cache
def _make_sc_gather():
    return functools.partial(
        pl.kernel,
        out_type=[
            jax.ShapeDtypeStruct((_B, _K), jnp.float32),
            jax.ShapeDtypeStruct((_B,), jnp.float32),
        ],
        mesh=plsc.VectorSubcoreMesh(core_axis_name="c", subcore_axis_name="s"),
        scratch_types=[
            pltpu.VMEM((_BPW,), jnp.int32),
            pltpu.VMEM((_BPW,), jnp.int32),
            pltpu.VMEM((_BPW,), jnp.int32),
            pltpu.VMEM((_BPW,), jnp.float32),
            pltpu.VMEM((_BPW,), jnp.float32),
            pltpu.VMEM((_BPW,), jnp.float32),
            pltpu.VMEM((_BPW,), jnp.float32),
            pltpu.VMEM((2, _CH, _K), jnp.float32),
            pltpu.VMEM((2, _CH, _K), jnp.float32),
            pltpu.SemaphoreType.DMA,
            pltpu.SemaphoreType.DMA,
            pltpu.SemaphoreType.DMA,
            pltpu.SemaphoreType.DMA,
            pltpu.SemaphoreType.DMA,
            pltpu.SemaphoreType.DMA,
        ],
    )(_sc_gather_body)


_BLK = 2048


_NCODE = 120


def _tc_mlp_body(inter_ref, code_ref, bias_ref,
                 dt_ref, wk_ref, yr_ref, w1a_ref, w1t_ref, b1_ref,
                 w2_ref, b2_ref, out_ref):
    f32 = jnp.float32
    inter = inter_ref[...]
    w1t = w1t_ref[...]
    pd = jnp.dot(dt_ref[...], w1t[0:10], preferred_element_type=f32)
    pw = jnp.dot(wk_ref[...], w1t[10:20], preferred_element_type=f32)
    py = jnp.dot(yr_ref[...], w1t[20:30], preferred_element_type=f32)
    c0 = lax.broadcasted_iota(jnp.int32, (_NCODE, 1), 0)
    e3 = (c0 % 3 == lax.broadcasted_iota(jnp.int32, (_NCODE, 3), 1)).astype(f32)
    e2 = ((c0 // 3) % 2
          == lax.broadcasted_iota(jnp.int32, (_NCODE, 2), 1)).astype(f32)
    e20 = (c0 // 6
           == lax.broadcasted_iota(jnp.int32, (_NCODE, 20), 1)).astype(f32)
    p = jnp.dot(e3, pd, preferred_element_type=f32) \
        + jnp.dot(e2, pw, preferred_element_type=f32) \
        + jnp.dot(e20, py, preferred_element_type=f32)
    code = code_ref[...]
    hot_t = (jnp.broadcast_to(code, (_NCODE, _BLK))
             == lax.broadcasted_iota(jnp.int32, (_NCODE, _BLK), 0)).astype(f32)
    acc = jnp.dot(inter, w1a_ref[...], preferred_element_type=f32)
    acc = acc + lax.dot_general(hot_t, p, (((0,), (0,)), ((), ())),
                                preferred_element_type=f32)
    h = jnp.maximum(acc + b1_ref[...], 0.0)
    w2 = w2_ref[...]
    parts = [
        lax.dot_general(w2, h[k * 128:(k + 1) * 128, :],
                        (((0,), (1,)), ((), ())), preferred_element_type=f32)
        for k in range(_BLK // 128)
    ]
    out_t = jnp.concatenate(parts, axis=0)
    out_ref[...] = out_t.reshape(_BLK) + b2_ref[0, 0] + bias_ref[...]


def _tc_mlp(inter, code, bvals, dt, wk, yr, w1a, w1t, b1, w2, b2):
    grid = (_B // _BLK,)
    row_spec = pl.BlockSpec((_BLK, _K), lambda i: (i, 0))
    vec_spec = pl.BlockSpec((_BLK,), lambda i: (i,))

    def full(a):
        return pl.BlockSpec(a.shape, lambda i: tuple(0 for _ in a.shape))

    return pl.pallas_call(
        _tc_mlp_body,
        grid=grid,
        in_specs=[row_spec, vec_spec, vec_spec,
                  full(dt), full(wk), full(yr), full(w1a), full(w1t),
                  full(b1), full(w2), full(b2)],
        out_specs=vec_spec,
        out_shape=jax.ShapeDtypeStruct((_B,), jnp.float32),
    )(inter, code, bvals, dt, wk, yr, w1a, w1t, b1, w2, b2)


def kernel(user_input, item_input, daytime_input, weekend_input, year_input,
           user_table, item_table, daytime_table, weekend_table, year_table,
           user_time_bias, W1, b1, W2, b2):
    ui = user_input.astype(jnp.int32)
    di = daytime_input.astype(jnp.int32)
    inter, bvals = _make_sc_gather()(
        ui, item_input.astype(jnp.int32), di, user_table, item_table,
        user_time_bias[:, 0], user_time_bias[:, 1], user_time_bias[:, 2])
    code = di + 3 * weekend_input.astype(jnp.int32) \
        + 6 * year_input.astype(jnp.int32)
    return _tc_mlp(
        inter, code, bvals,
        daytime_table, weekend_table, year_table,
        W1[0:_K], W1[_K:], b1.reshape(1, -1), W2, b2.reshape(1, 1))
